# baseline (speedup 1.0000x reference)
import functools

import jax
import jax.numpy as jnp
from jax import lax
from jax.experimental import pallas as pl
from jax.experimental.pallas import tpu as pltpu
from jax.experimental.pallas import tpu_sc as plsc

NUM_ENTITIES = 1000000
NUM_RELATIONS = 1000
D = 64
B = 16384

NC = 2
NS = 16
L = 16
NW = NC * NS
BPW = B // NW
GCHUNK = 128

_mesh = plsc.VectorSubcoreMesh(core_axis_name="c", subcore_axis_name="s")


@functools.partial(
    pl.kernel,
    mesh=_mesh,
    out_type=jax.ShapeDtypeStruct((B,), jnp.float32),
    compiler_params=pltpu.CompilerParams(needs_layout_passes=False,
                                         use_tc_tiling_on_sc=False),
    scratch_types=[
        pltpu.VMEM((BPW,), jnp.int32),
        pltpu.VMEM((BPW,), jnp.int32),
        pltpu.VMEM((BPW,), jnp.int32),
        pltpu.VMEM((BPW, D), jnp.float32),
        pltpu.VMEM((BPW, D), jnp.float32),
        pltpu.VMEM((BPW, D), jnp.float32),
        pltpu.VMEM((BPW,), jnp.float32),
        pltpu.SemaphoreType.DMA,
    ],
)
def _distmult_sc(node_hbm, edge_hbm, e_hbm, p_hbm, u_hbm, out_hbm,
                 e_idx, p_idx, u_idx, e_rows, p_rows, u_rows, out_v, sem):
    wid = lax.axis_index("s") * NC + lax.axis_index("c")
    base = wid * BPW

    pltpu.sync_copy(e_hbm.at[pl.ds(base, BPW)], e_idx)
    pltpu.sync_copy(p_hbm.at[pl.ds(base, BPW)], p_idx)
    pltpu.sync_copy(u_hbm.at[pl.ds(base, BPW)], u_idx)

    copies = []
    for j in range(BPW // GCHUNK):
        sl = pl.ds(j * GCHUNK, GCHUNK)
        copies.append(pltpu.async_copy(node_hbm.at[e_idx.at[sl]], e_rows.at[sl], sem))
        copies.append(pltpu.async_copy(node_hbm.at[u_idx.at[sl]], u_rows.at[sl], sem))
        copies.append(pltpu.async_copy(edge_hbm.at[p_idx.at[sl]], p_rows.at[sl], sem))
    for c in copies:
        c.wait()

    lane = lax.iota(jnp.int32, L)

    def group(g, carry):
        row0 = g * L
        res = jnp.zeros((L,), jnp.float32)
        for i in range(L):
            b = row0 + i
            acc = jnp.zeros((L,), jnp.float32)
            for c in range(D // L):
                sl = pl.ds(c * L, L)
                acc = acc + e_rows[b, sl] * p_rows[b, sl] * u_rows[b, sl]
            res = jnp.where(lane == i, jnp.sum(acc), res)
        out_v[pl.ds(row0, L)] = res
        return carry

    lax.fori_loop(0, BPW // L, group, 0)

    pltpu.sync_copy(out_v, out_hbm.at[pl.ds(base, BPW)])


def kernel(node_embeddings, edge_embeddings, e_idc, p_idc, u_idc):
    return _distmult_sc(node_embeddings, edge_embeddings,
                        e_idc.astype(jnp.int32), p_idc.astype(jnp.int32),
                        u_idc.astype(jnp.int32))
